# Initial kernel scaffold; baseline (speedup 1.0000x reference)
import jax
import jax.numpy as jnp
from jax.experimental import pallas as pl


def kernel(x, edge_index, edge_attr, params):
    raise NotImplementedError("write your pallas kernel here")


# speedup vs baseline: 4.2075x; 4.2075x over previous
import functools

import jax
import jax.numpy as jnp
from jax import lax
from jax.experimental import pallas as pl
from jax.experimental.pallas import tpu as pltpu
from jax.experimental.pallas import tpu_sc as plsc

N = 10000
E = 320000
D = 128
H = 16
NC = 2
NS = 16
NW = NC * NS
CH = 80
NCHT = E // CH
GPC = CH // H
BR = 40
NBLK = N // BR


def _sc_edge_body(src_hbm, dst_hbm, ea_hbm, u_hbm, v_hbm, xb_hbm, prm_hbm,
                  out_hbm,
                  aggr_sh, src_v, dst_v, ea_v, u_rows, v_rows, x_rows,
                  w_v, prm_v, zbuf, sem_u, sem_v, sem_x):
    cid = lax.axis_index("c")
    sid = lax.axis_index("s")
    wid = cid * NS + sid

    for r in range(BR):
        for c in range(D // H):
            zbuf[r, pl.ds(c * H, H)] = jnp.zeros((H,), jnp.float32)
    nblk = 15 + jnp.where(sid < NBLK - 15 * NS, 1, 0)

    def zero_blk(i, carry):
        b = i * NS + sid
        pltpu.sync_copy(zbuf, aggr_sh.at[pl.ds(b * BR, BR)])
        return carry

    lax.fori_loop(0, nblk, zero_blk, 0, unroll=False)
    plsc.subcore_barrier()

    pltpu.sync_copy(prm_hbm, prm_v)
    aev = prm_v[0, :]
    bav = prm_v[1, :]
    bvv = prm_v[2, :]
    bB = prm_v[3, :][0]

    def chunk(i, carry):
        base = pl.multiple_of((wid + i * NW) * CH, CH)
        pltpu.sync_copy(src_hbm.at[pl.ds(base, CH)], src_v)
        pltpu.sync_copy(dst_hbm.at[pl.ds(base, CH)], dst_v)
        pltpu.sync_copy(ea_hbm.at[pl.ds(base, CH)], ea_v)
        cu = pltpu.async_copy(u_hbm.at[dst_v], u_rows, sem_u)
        cv = pltpu.async_copy(v_hbm.at[src_v], v_rows, sem_v)
        cx = pltpu.async_copy(xb_hbm.at[src_v], x_rows, sem_x)
        cu.wait()
        cv.wait()
        cx.wait()

        for g in range(GPC):
            ea_g = ea_v[pl.ds(g * H, H)]
            sg = src_v[pl.ds(g * H, H)]
            dg = dst_v[pl.ds(g * H, H)]
            ridx = jnp.arange(H, dtype=jnp.int32) + (g * H)
            acc = jnp.full((H,), 0.0, jnp.float32) + bB
            for l in range(H):
                cidx = jnp.full((H,), l, jnp.int32)
                ucol = plsc.load_gather(u_rows, [ridx, cidx])
                vcol = plsc.load_gather(v_rows, [ridx, cidx])
                h = jnp.maximum(ucol + vcol + ea_g * aev[l] + bav[l], 0.0)
                acc = acc + h * bvv[l]
            w = 1.0 / (1.0 + jnp.exp(-acc))
            w = jnp.where(sg != dg, w, jnp.zeros((H,), jnp.float32))
            w_v[pl.ds(g * H, H)] = w

        def scale(j, c2):
            wj = plsc.load_gather(w_v, [jnp.full((H,), j, jnp.int32)])
            for c in range(D // H):
                x_rows[j, pl.ds(c * H, H)] = x_rows[j, pl.ds(c * H, H)] * wj
            return c2

        lax.fori_loop(0, CH, scale, 0, unroll=False)
        pltpu.sync_copy(x_rows, aggr_sh.at[dst_v], add=True)
        return carry

    lax.fori_loop(0, NCHT // NW, chunk, 0, unroll=False)
    plsc.subcore_barrier()

    def read_blk(i, carry):
        b = i * NS + sid
        pltpu.sync_copy(aggr_sh.at[pl.ds(b * BR, BR)],
                        out_hbm.at[cid, pl.ds(b * BR, BR)])
        return carry

    lax.fori_loop(0, nblk, read_blk, 0, unroll=False)


_sc_edge = functools.partial(
    pl.kernel,
    out_type=jax.ShapeDtypeStruct((NC, N, D), jnp.float32),
    mesh=plsc.VectorSubcoreMesh(core_axis_name="c", subcore_axis_name="s",
                                num_cores=NC, num_subcores=NS),
    compiler_params=pltpu.CompilerParams(needs_layout_passes=False),
    scratch_types=[
        pltpu.VMEM_SHARED((N, D), jnp.float32),
        pltpu.VMEM((CH,), jnp.int32),
        pltpu.VMEM((CH,), jnp.int32),
        pltpu.VMEM((CH,), jnp.float32),
        pltpu.VMEM((CH, D), jnp.float32),
        pltpu.VMEM((CH, D), jnp.float32),
        pltpu.VMEM((CH, D), jnp.float32),
        pltpu.VMEM((CH,), jnp.float32),
        pltpu.VMEM((4, H), jnp.float32),
        pltpu.VMEM((BR, D), jnp.float32),
        pltpu.SemaphoreType.DMA,
        pltpu.SemaphoreType.DMA,
        pltpu.SemaphoreType.DMA,
    ],
)(_sc_edge_body)


def _pre0_body(x_ref, ai_ref, aj_ref, ba_ref, bv_ref, bb_ref,
               u_ref, v_ref, wl_ref):
    xb = x_ref[...]
    u = jnp.dot(xb, ai_ref[...], preferred_element_type=jnp.float32)
    v = jnp.dot(xb, aj_ref[...], preferred_element_type=jnp.float32)
    u_ref[...] = u
    v_ref[...] = v
    hl = jnp.maximum(u + v + ba_ref[...], 0.0)
    logit = jnp.sum(hl * bv_ref[...], axis=1, keepdims=True) + bb_ref[...]
    wl_ref[...] = 1.0 / (1.0 + jnp.exp(-logit))


def _pre_bn_body(x_ref, g_ref, gb_ref, ai_ref, aj_ref, ba_ref, bv_ref, bb_ref,
                 xb_ref, u_ref, v_ref, wl_ref):
    x = x_ref[...]
    m = jnp.mean(x, axis=0, keepdims=True)
    xm = x - m
    var = jnp.mean(xm * xm, axis=0, keepdims=True)
    xb = xm / jnp.sqrt(var + 1e-5) * g_ref[...] + gb_ref[...]
    xb_ref[...] = xb
    u = jnp.dot(xb, ai_ref[...], preferred_element_type=jnp.float32)
    v = jnp.dot(xb, aj_ref[...], preferred_element_type=jnp.float32)
    u_ref[...] = u
    v_ref[...] = v
    hl = jnp.maximum(u + v + ba_ref[...], 0.0)
    logit = jnp.sum(hl * bv_ref[...], axis=1, keepdims=True) + bb_ref[...]
    wl_ref[...] = 1.0 / (1.0 + jnp.exp(-logit))


def _post_body(xb_ref, p_ref, wl_ref, wu_ref, wlw_ref, b_ref, out_ref):
    xb = xb_ref[...]
    aggr = p_ref[0] + p_ref[1] + wl_ref[...] * xb
    out_ref[...] = jnp.maximum(
        jnp.dot(xb, wu_ref[...], preferred_element_type=jnp.float32)
        + jnp.dot(aggr, wlw_ref[...], preferred_element_type=jnp.float32)
        + b_ref[...], 0.0)


def _heads1_body(emb_ref, few_ref, feb_ref, w1_ref, b1_ref, w2_ref, b2_ref,
                 wf_ref, bf_ref, x_ref, f_ref, xm_ref):
    x = (jnp.dot(emb_ref[...], few_ref[...],
                 preferred_element_type=jnp.float32) + feb_ref[...])
    x_ref[...] = x
    h = jnp.maximum(jnp.dot(x, w1_ref[...],
                            preferred_element_type=jnp.float32) + b1_ref[...],
                    0.0)
    h = jnp.maximum(jnp.dot(h, w2_ref[...],
                            preferred_element_type=jnp.float32) + b2_ref[...],
                    0.0)
    f_ref[...] = (jnp.dot(h, wf_ref[...],
                          preferred_element_type=jnp.float32) + bf_ref[...])
    xm_ref[...] = jnp.mean(x, axis=0, keepdims=True)


def _heads2_body(x_ref, t_ref, w1b_ref, w2_ref, b2_ref, wf_ref, bf_ref,
                 s_ref):
    h1 = jnp.maximum(jnp.dot(x_ref[...], w1b_ref[...],
                             preferred_element_type=jnp.float32) + t_ref[...],
                     0.0)
    h2 = jnp.maximum(jnp.dot(h1, w2_ref[...],
                             preferred_element_type=jnp.float32) + b2_ref[...],
                     0.0)
    s_ref[...] = (jnp.dot(h2, wf_ref[...],
                          preferred_element_type=jnp.float32) + bf_ref[...])


def _call_tc(body, out_shapes, *args):
    return pl.pallas_call(
        body,
        out_shape=[jax.ShapeDtypeStruct(s, jnp.float32) for s in out_shapes],
    )(*args)


def _mlp_vec(p, h):
    for lp in p["layers"]:
        h = jnp.maximum(h @ lp["w"] + lp["b"], 0.0)
    return h @ p["final"]["w"] + p["final"]["b"]


def kernel(x, edge_index, edge_attr, params):
    src = edge_index[0]
    dst = edge_index[1]
    ea = edge_attr

    emb = x
    for li, p in enumerate(params["gnn"]):
        a = p["linA"]["w"]
        pad = jnp.zeros((D, D - H), jnp.float32)
        ai = jnp.concatenate([a[:D], pad], axis=1)
        aj = jnp.concatenate([a[D:2 * D], pad], axis=1)
        zpad = jnp.zeros((1, D - H), jnp.float32)
        ba = jnp.concatenate([p["linA"]["b"].reshape(1, H), zpad], axis=1)
        bv = jnp.concatenate([p["linB"]["w"].reshape(1, H), zpad], axis=1)
        bb = p["linB"]["b"].reshape(1, 1)
        prm = jnp.stack([a[2 * D], p["linA"]["b"], p["linB"]["w"][:, 0],
                         jnp.full((H,), p["linB"]["b"][0], jnp.float32)])
        if li == 0:
            xb = emb
            u, v, wl = _call_tc(_pre0_body, [(N, D), (N, D), (N, 1)],
                                xb, ai, aj, ba, bv, bb)
        else:
            g = p["bn"]["g"].reshape(1, D)
            gb = p["bn"]["b"].reshape(1, D)
            xb, u, v, wl = _call_tc(_pre_bn_body,
                                    [(N, D), (N, D), (N, D), (N, 1)],
                                    emb, g, gb, ai, aj, ba, bv, bb)
        partials = _sc_edge(src, dst, ea, u, v, xb, prm)
        w = p["lin"]["w"]
        emb, = _call_tc(_post_body, [(N, D)],
                        xb, partials, wl, w[:D], w[D:], p["lin"]["b"].reshape(1, D))

    mf = params["mf"]
    X, f, xm = _call_tc(
        _heads1_body, [(N, D), (N, 1), (1, D)],
        emb, params["final_emb"]["w"], params["final_emb"]["b"].reshape(1, D),
        mf["layers"][0]["w"], mf["layers"][0]["b"].reshape(1, D),
        mf["layers"][1]["w"], mf["layers"][1]["b"].reshape(1, D),
        mf["final"]["w"], mf["final"]["b"].reshape(1, 1))
    f = f[:, 0]

    key = jax.random.key(123)
    p1v = jax.nn.softmax(f, axis=0)
    p1v = p1v * jnp.ones((N,), jnp.float32).at[N - 9:].set(0.0)
    a1 = jax.random.categorical(jax.random.fold_in(key, 0),
                                jnp.log(p1v)).astype(jnp.int32)
    p1 = p1v[a1]

    ms = params["ms"]
    xa1 = X[a1]
    t = (xa1 @ ms["layers"][0]["w"][:D] + ms["layers"][0]["b"]).reshape(1, D)
    s, = _call_tc(_heads2_body, [(N, 1)],
                  X, t, ms["layers"][0]["w"][D:],
                  ms["layers"][1]["w"], ms["layers"][1]["b"].reshape(1, D),
                  ms["final"]["w"], ms["final"]["b"].reshape(1, 1))
    s = s[:, 0]
    p2v = jax.nn.softmax(s, axis=0).at[a1].set(0.0)
    a2 = jax.random.categorical(jax.random.fold_in(key, 1),
                                jnp.log(p2v)).astype(jnp.int32)
    p2 = p2v[a2]

    xc = jnp.concatenate([xa1, X[a2]], axis=0)
    el = _mlp_vec(params["me"], xc)
    epv = jax.nn.softmax(el, axis=0)
    ae = jax.random.categorical(jax.random.fold_in(key, 2),
                                jnp.log(epv)).astype(jnp.int32)
    pe = epv[ae]

    pstop = 1.0 / (1.0 + jnp.exp(-_mlp_vec(params["mt"], xm[0])))
    pstop = pstop[0]
    ast = jax.random.bernoulli(jax.random.fold_in(key, 3),
                               pstop).astype(jnp.int32)
    ps = jnp.where(ast == 0, 1.0 - pstop, pstop)

    acts = jnp.stack([a1, a2, ae, ast]).astype(jnp.int32)
    probs = jnp.stack([p1, p2, pe, ps])
    return acts, probs


# speedup vs baseline: 4.2207x; 1.0031x over previous
import functools

import jax
import jax.numpy as jnp
from jax import lax
from jax.experimental import pallas as pl
from jax.experimental.pallas import tpu as pltpu
from jax.experimental.pallas import tpu_sc as plsc

N = 10000
E = 320000
D = 128
H = 16
NC = 2
NS = 16
NW = NC * NS
CH = 80
NCHT = E // CH
GPC = CH // H
BR = 40
NBLK = N // BR


def _sc_edge_body(src_hbm, dst_hbm, ea_hbm, u_hbm, v_hbm, xb_hbm, prm_hbm,
                  out_hbm,
                  aggr_sh, src_v, dst_v, ea_v, u_rows, v_rows, x_rows,
                  w_v, prm_v, zbuf, sem_u, sem_v, sem_x):
    cid = lax.axis_index("c")
    sid = lax.axis_index("s")
    wid = cid * NS + sid

    for r in range(BR):
        for c in range(D // H):
            zbuf[r, pl.ds(c * H, H)] = jnp.zeros((H,), jnp.float32)
    nblk = 15 + jnp.where(sid < NBLK - 15 * NS, 1, 0)

    def zero_blk(i, carry):
        b = i * NS + sid
        pltpu.sync_copy(zbuf, aggr_sh.at[pl.ds(b * BR, BR)])
        return carry

    lax.fori_loop(0, nblk, zero_blk, 0, unroll=False)
    plsc.subcore_barrier()

    pltpu.sync_copy(prm_hbm, prm_v)
    aev = prm_v[0, :]
    bav = prm_v[1, :]
    bvv = prm_v[2, :]
    bB = prm_v[3, :][0]

    def chunk(i, carry):
        base = pl.multiple_of((wid + i * NW) * CH, CH)
        pltpu.sync_copy(src_hbm.at[pl.ds(base, CH)], src_v)
        pltpu.sync_copy(dst_hbm.at[pl.ds(base, CH)], dst_v)
        pltpu.sync_copy(ea_hbm.at[pl.ds(base, CH)], ea_v)
        cu = pltpu.async_copy(u_hbm.at[dst_v], u_rows, sem_u)
        cv = pltpu.async_copy(v_hbm.at[src_v], v_rows, sem_v)
        cx = pltpu.async_copy(xb_hbm.at[src_v], x_rows, sem_x)
        cu.wait()
        cv.wait()

        for g in range(GPC):
            ea_g = ea_v[pl.ds(g * H, H)]
            sg = src_v[pl.ds(g * H, H)]
            dg = dst_v[pl.ds(g * H, H)]
            ridx = jnp.arange(H, dtype=jnp.int32) + (g * H)
            acc = jnp.full((H,), 0.0, jnp.float32) + bB
            for l in range(H):
                cidx = jnp.full((H,), l, jnp.int32)
                ucol = plsc.load_gather(u_rows, [ridx, cidx])
                vcol = plsc.load_gather(v_rows, [ridx, cidx])
                h = jnp.maximum(ucol + vcol + ea_g * aev[l] + bav[l], 0.0)
                acc = acc + h * bvv[l]
            w = 1.0 / (1.0 + jnp.exp(-acc))
            w = jnp.where(sg != dg, w, jnp.zeros((H,), jnp.float32))
            w_v[pl.ds(g * H, H)] = w

        cx.wait()

        def scale(j, c2):
            wj = plsc.load_gather(w_v, [jnp.full((H,), j, jnp.int32)])
            for c in range(D // H):
                x_rows[j, pl.ds(c * H, H)] = x_rows[j, pl.ds(c * H, H)] * wj
            return c2

        lax.fori_loop(0, CH, scale, 0, unroll=False)
        pltpu.sync_copy(x_rows, aggr_sh.at[dst_v], add=True)
        return carry

    lax.fori_loop(0, NCHT // NW, chunk, 0, unroll=False)
    plsc.subcore_barrier()

    def read_blk(i, carry):
        b = i * NS + sid
        pltpu.sync_copy(aggr_sh.at[pl.ds(b * BR, BR)],
                        out_hbm.at[cid, pl.ds(b * BR, BR)])
        return carry

    lax.fori_loop(0, nblk, read_blk, 0, unroll=False)


_sc_edge = functools.partial(
    pl.kernel,
    out_type=jax.ShapeDtypeStruct((NC, N, D), jnp.float32),
    mesh=plsc.VectorSubcoreMesh(core_axis_name="c", subcore_axis_name="s",
                                num_cores=NC, num_subcores=NS),
    compiler_params=pltpu.CompilerParams(needs_layout_passes=False),
    scratch_types=[
        pltpu.VMEM_SHARED((N, D), jnp.float32),
        pltpu.VMEM((CH,), jnp.int32),
        pltpu.VMEM((CH,), jnp.int32),
        pltpu.VMEM((CH,), jnp.float32),
        pltpu.VMEM((CH, D), jnp.float32),
        pltpu.VMEM((CH, D), jnp.float32),
        pltpu.VMEM((CH, D), jnp.float32),
        pltpu.VMEM((CH,), jnp.float32),
        pltpu.VMEM((4, H), jnp.float32),
        pltpu.VMEM((BR, D), jnp.float32),
        pltpu.SemaphoreType.DMA,
        pltpu.SemaphoreType.DMA,
        pltpu.SemaphoreType.DMA,
    ],
)(_sc_edge_body)


def _pre0_body(x_ref, ai_ref, aj_ref, ba_ref, bv_ref, bb_ref,
               u_ref, v_ref, wl_ref):
    xb = x_ref[...]
    u = jnp.dot(xb, ai_ref[...], preferred_element_type=jnp.float32)
    v = jnp.dot(xb, aj_ref[...], preferred_element_type=jnp.float32)
    u_ref[...] = u
    v_ref[...] = v
    hl = jnp.maximum(u + v + ba_ref[...], 0.0)
    logit = jnp.sum(hl * bv_ref[...], axis=1, keepdims=True) + bb_ref[...]
    wl_ref[...] = 1.0 / (1.0 + jnp.exp(-logit))


def _pre_bn_body(x_ref, g_ref, gb_ref, ai_ref, aj_ref, ba_ref, bv_ref, bb_ref,
                 xb_ref, u_ref, v_ref, wl_ref):
    x = x_ref[...]
    m = jnp.mean(x, axis=0, keepdims=True)
    xm = x - m
    var = jnp.mean(xm * xm, axis=0, keepdims=True)
    xb = xm / jnp.sqrt(var + 1e-5) * g_ref[...] + gb_ref[...]
    xb_ref[...] = xb
    u = jnp.dot(xb, ai_ref[...], preferred_element_type=jnp.float32)
    v = jnp.dot(xb, aj_ref[...], preferred_element_type=jnp.float32)
    u_ref[...] = u
    v_ref[...] = v
    hl = jnp.maximum(u + v + ba_ref[...], 0.0)
    logit = jnp.sum(hl * bv_ref[...], axis=1, keepdims=True) + bb_ref[...]
    wl_ref[...] = 1.0 / (1.0 + jnp.exp(-logit))


def _post_body(xb_ref, p_ref, wl_ref, wu_ref, wlw_ref, b_ref, out_ref):
    xb = xb_ref[...]
    aggr = p_ref[0] + p_ref[1] + wl_ref[...] * xb
    out_ref[...] = jnp.maximum(
        jnp.dot(xb, wu_ref[...], preferred_element_type=jnp.float32)
        + jnp.dot(aggr, wlw_ref[...], preferred_element_type=jnp.float32)
        + b_ref[...], 0.0)


def _heads1_body(emb_ref, few_ref, feb_ref, w1_ref, b1_ref, w2_ref, b2_ref,
                 wf_ref, bf_ref, x_ref, f_ref, xm_ref):
    x = (jnp.dot(emb_ref[...], few_ref[...],
                 preferred_element_type=jnp.float32) + feb_ref[...])
    x_ref[...] = x
    h = jnp.maximum(jnp.dot(x, w1_ref[...],
                            preferred_element_type=jnp.float32) + b1_ref[...],
                    0.0)
    h = jnp.maximum(jnp.dot(h, w2_ref[...],
                            preferred_element_type=jnp.float32) + b2_ref[...],
                    0.0)
    f_ref[...] = (jnp.dot(h, wf_ref[...],
                          preferred_element_type=jnp.float32) + bf_ref[...])
    xm_ref[...] = jnp.mean(x, axis=0, keepdims=True)


def _heads2_body(x_ref, t_ref, w1b_ref, w2_ref, b2_ref, wf_ref, bf_ref,
                 s_ref):
    h1 = jnp.maximum(jnp.dot(x_ref[...], w1b_ref[...],
                             preferred_element_type=jnp.float32) + t_ref[...],
                     0.0)
    h2 = jnp.maximum(jnp.dot(h1, w2_ref[...],
                             preferred_element_type=jnp.float32) + b2_ref[...],
                     0.0)
    s_ref[...] = (jnp.dot(h2, wf_ref[...],
                          preferred_element_type=jnp.float32) + bf_ref[...])


def _call_tc(body, out_shapes, *args):
    return pl.pallas_call(
        body,
        out_shape=[jax.ShapeDtypeStruct(s, jnp.float32) for s in out_shapes],
    )(*args)


def _mlp_vec(p, h):
    for lp in p["layers"]:
        h = jnp.maximum(h @ lp["w"] + lp["b"], 0.0)
    return h @ p["final"]["w"] + p["final"]["b"]


def kernel(x, edge_index, edge_attr, params):
    src = edge_index[0]
    dst = edge_index[1]
    ea = edge_attr

    emb = x
    for li, p in enumerate(params["gnn"]):
        a = p["linA"]["w"]
        pad = jnp.zeros((D, D - H), jnp.float32)
        ai = jnp.concatenate([a[:D], pad], axis=1)
        aj = jnp.concatenate([a[D:2 * D], pad], axis=1)
        zpad = jnp.zeros((1, D - H), jnp.float32)
        ba = jnp.concatenate([p["linA"]["b"].reshape(1, H), zpad], axis=1)
        bv = jnp.concatenate([p["linB"]["w"].reshape(1, H), zpad], axis=1)
        bb = p["linB"]["b"].reshape(1, 1)
        prm = jnp.stack([a[2 * D], p["linA"]["b"], p["linB"]["w"][:, 0],
                         jnp.full((H,), p["linB"]["b"][0], jnp.float32)])
        if li == 0:
            xb = emb
            u, v, wl = _call_tc(_pre0_body, [(N, D), (N, D), (N, 1)],
                                xb, ai, aj, ba, bv, bb)
        else:
            g = p["bn"]["g"].reshape(1, D)
            gb = p["bn"]["b"].reshape(1, D)
            xb, u, v, wl = _call_tc(_pre_bn_body,
                                    [(N, D), (N, D), (N, D), (N, 1)],
                                    emb, g, gb, ai, aj, ba, bv, bb)
        partials = _sc_edge(src, dst, ea, u, v, xb, prm)
        w = p["lin"]["w"]
        emb, = _call_tc(_post_body, [(N, D)],
                        xb, partials, wl, w[:D], w[D:], p["lin"]["b"].reshape(1, D))

    mf = params["mf"]
    X, f, xm = _call_tc(
        _heads1_body, [(N, D), (N, 1), (1, D)],
        emb, params["final_emb"]["w"], params["final_emb"]["b"].reshape(1, D),
        mf["layers"][0]["w"], mf["layers"][0]["b"].reshape(1, D),
        mf["layers"][1]["w"], mf["layers"][1]["b"].reshape(1, D),
        mf["final"]["w"], mf["final"]["b"].reshape(1, 1))
    f = f[:, 0]

    key = jax.random.key(123)
    p1v = jax.nn.softmax(f, axis=0)
    p1v = p1v * jnp.ones((N,), jnp.float32).at[N - 9:].set(0.0)
    a1 = jax.random.categorical(jax.random.fold_in(key, 0),
                                jnp.log(p1v)).astype(jnp.int32)
    p1 = p1v[a1]

    ms = params["ms"]
    xa1 = X[a1]
    t = (xa1 @ ms["layers"][0]["w"][:D] + ms["layers"][0]["b"]).reshape(1, D)
    s, = _call_tc(_heads2_body, [(N, 1)],
                  X, t, ms["layers"][0]["w"][D:],
                  ms["layers"][1]["w"], ms["layers"][1]["b"].reshape(1, D),
                  ms["final"]["w"], ms["final"]["b"].reshape(1, 1))
    s = s[:, 0]
    p2v = jax.nn.softmax(s, axis=0).at[a1].set(0.0)
    a2 = jax.random.categorical(jax.random.fold_in(key, 1),
                                jnp.log(p2v)).astype(jnp.int32)
    p2 = p2v[a2]

    xc = jnp.concatenate([xa1, X[a2]], axis=0)
    el = _mlp_vec(params["me"], xc)
    epv = jax.nn.softmax(el, axis=0)
    ae = jax.random.categorical(jax.random.fold_in(key, 2),
                                jnp.log(epv)).astype(jnp.int32)
    pe = epv[ae]

    pstop = 1.0 / (1.0 + jnp.exp(-_mlp_vec(params["mt"], xm[0])))
    pstop = pstop[0]
    ast = jax.random.bernoulli(jax.random.fold_in(key, 3),
                               pstop).astype(jnp.int32)
    ps = jnp.where(ast == 0, 1.0 - pstop, pstop)

    acts = jnp.stack([a1, a2, ae, ast]).astype(jnp.int32)
    probs = jnp.stack([p1, p2, pe, ps])
    return acts, probs


# speedup vs baseline: 7.2051x; 1.7071x over previous
import functools

import jax
import jax.numpy as jnp
from jax import lax
from jax.experimental import pallas as pl
from jax.experimental.pallas import tpu as pltpu
from jax.experimental.pallas import tpu_sc as plsc

N = 10000
E = 320000
D = 128
H = 16
NC = 2
NS = 16
NW = NC * NS
CH = 32
NCHT = E // CH
GPC = CH // H
BR = 40
NBLK = N // BR


NCH0 = NCHT // NW
NXW = NCHT - NCH0 * NW
NPAIR = NCH0 // 2
W3 = 3 * CH


def _sc_edge_body(eidx_hbm, u_hbm, v_hbm, xb_hbm, prm_hbm,
                  out_hbm,
                  aggr_sh, idxA, idxB, srcA, srcB, dstA, dstB, dstAs, dstBs,
                  uA, uB, vA, vB, xA, xB, w_v, prm_v,
                  semIA, semIB, semUA, semUB, semVA, semVB,
                  semXA, semXB, semSA, semSB):
    cid = lax.axis_index("c")
    sid = lax.axis_index("s")
    wid = cid * NS + sid
    nchl = NCH0 - 1 + jnp.where(wid < NXW, 1, 0)

    for r in range(BR):
        for c in range(D // H):
            xA[r, pl.ds(c * H, H)] = jnp.zeros((H,), jnp.float32)
    nblk = 15 + jnp.where(sid < NBLK - 15 * NS, 1, 0)

    def zero_blk(i, carry):
        b = i * NS + sid
        pltpu.sync_copy(xA.at[pl.ds(0, BR)], aggr_sh.at[pl.ds(b * BR, BR)])
        return carry

    lax.fori_loop(0, nblk, zero_blk, 0, unroll=False)
    plsc.subcore_barrier()

    pltpu.sync_copy(prm_hbm, prm_v)
    aev = prm_v[0, :]
    bav = prm_v[1, :]
    bvv = prm_v[2, :]
    bB = prm_v[3, :][0]

    def idx_off(k):
        return (wid + k * NW) * W3

    def extract(idx, srcb, dstb):
        def body(g, c2):
            srcb[pl.ds(g * H, H)] = idx[pl.ds(g * H, H)]
            dstb[pl.ds(g * H, H)] = idx[pl.ds(CH + g * H, H)]
            return c2

        lax.fori_loop(0, GPC, body, 0, unroll=False)

    def group_compute(idx, srcb, dstb, ub, vb):
        def body(g, c2):
            ea_g = plsc.bitcast(idx[pl.ds(2 * CH + g * H, H)], jnp.float32)
            sg = srcb[pl.ds(g * H, H)]
            dg = dstb[pl.ds(g * H, H)]
            ridx = jnp.arange(H, dtype=jnp.int32) + g * H
            acc = jnp.full((H,), 0.0, jnp.float32) + bB
            for l in range(H):
                cidx = jnp.full((H,), l, jnp.int32)
                ucol = plsc.load_gather(ub, [ridx, cidx])
                vcol = plsc.load_gather(vb, [ridx, cidx])
                h = jnp.maximum(ucol + vcol + ea_g * aev[l] + bav[l], 0.0)
                acc = acc + h * bvv[l]
            w = 1.0 / (1.0 + jnp.exp(-acc))
            w = jnp.where(sg != dg, w, jnp.zeros((H,), jnp.float32))
            w_v[pl.ds(g * H, H)] = w
            return c2

        lax.fori_loop(0, GPC, body, 0, unroll=False)

    def scale_rows(xbuf):
        def body(j, c2):
            wj = plsc.load_gather(w_v, [jnp.full((H,), j, jnp.int32)])
            for c in range(D // H):
                xbuf[j, pl.ds(c * H, H)] = xbuf[j, pl.ds(c * H, H)] * wj
            return c2

        lax.fori_loop(0, CH, body, 0, unroll=False)

    def copy_dst(dstb, dsts):
        def body(g, c2):
            dsts[pl.ds(g * H, H)] = dstb[pl.ds(g * H, H)]
            return c2

        lax.fori_loop(0, GPC, body, 0, unroll=False)

    pltpu.sync_copy(eidx_hbm.at[pl.ds(pl.multiple_of(idx_off(0), W3), W3)],
                    idxA)
    extract(idxA, srcA, dstA)
    pltpu.async_copy(u_hbm.at[dstA], uA, semUA)
    pltpu.async_copy(v_hbm.at[srcA], vA, semVA)
    pltpu.async_copy(xb_hbm.at[srcA], xA, semXA)
    pltpu.async_copy(eidx_hbm.at[pl.ds(pl.multiple_of(idx_off(1), W3), W3)],
                     idxB, semIB)

    def pair(i, carry):
        k0 = 2 * i
        pltpu.make_async_copy(
            eidx_hbm.at[pl.ds(pl.multiple_of(idx_off(k0 + 1), W3), W3)],
            idxB, semIB).wait()
        extract(idxB, srcB, dstB)
        pltpu.async_copy(u_hbm.at[dstB], uB, semUB)
        pltpu.async_copy(v_hbm.at[srcB], vB, semVB)
        pltpu.async_copy(xb_hbm.at[srcB], xB, semXB)
        pltpu.make_async_copy(u_hbm.at[dstA], uA, semUA).wait()
        pltpu.make_async_copy(v_hbm.at[srcA], vA, semVA).wait()
        group_compute(idxA, srcA, dstA, uA, vA)
        kp = jnp.minimum(k0 + 2, nchl)
        cIA = pltpu.async_copy(eidx_hbm.at[pl.ds(idx_off(kp), W3)],
                               idxA, semIA)
        pltpu.make_async_copy(xb_hbm.at[srcA], xA, semXA).wait()
        scale_rows(xA)
        copy_dst(dstA, dstAs)
        sA = pltpu.async_copy(xA, aggr_sh.at[dstAs], semSA, add=True)
        cIA.wait()
        extract(idxA, srcA, dstA)
        pltpu.async_copy(u_hbm.at[dstA], uA, semUA)
        pltpu.async_copy(v_hbm.at[srcA], vA, semVA)
        pltpu.make_async_copy(u_hbm.at[dstB], uB, semUB).wait()
        pltpu.make_async_copy(v_hbm.at[srcB], vB, semVB).wait()
        group_compute(idxB, srcB, dstB, uB, vB)
        kq = jnp.minimum(k0 + 3, nchl)
        pltpu.async_copy(eidx_hbm.at[pl.ds(idx_off(kq), W3)], idxB, semIB)
        pltpu.make_async_copy(xb_hbm.at[srcB], xB, semXB).wait()
        scale_rows(xB)
        copy_dst(dstB, dstBs)
        sA.wait()
        pltpu.async_copy(xb_hbm.at[srcA], xA, semXA)
        sB = pltpu.async_copy(xB, aggr_sh.at[dstBs], semSB, add=True)
        sB.wait()
        return carry

    lax.fori_loop(0, NPAIR, pair, 0, unroll=False)

    pltpu.make_async_copy(eidx_hbm.at[pl.ds(idx_off(nchl), W3)],
                          idxB, semIB).wait()
    pltpu.make_async_copy(u_hbm.at[dstA], uA, semUA).wait()
    pltpu.make_async_copy(v_hbm.at[srcA], vA, semVA).wait()
    pltpu.make_async_copy(xb_hbm.at[srcA], xA, semXA).wait()

    @pl.when(wid < NXW)
    def _leftover():
        group_compute(idxA, srcA, dstA, uA, vA)
        scale_rows(xA)
        pltpu.sync_copy(xA, aggr_sh.at[dstA], add=True)

    plsc.subcore_barrier()

    def read_blk(i, carry):
        b = i * NS + sid
        pltpu.sync_copy(aggr_sh.at[pl.ds(b * BR, BR)],
                        out_hbm.at[cid, pl.ds(b * BR, BR)])
        return carry

    lax.fori_loop(0, nblk, read_blk, 0, unroll=False)


_sc_edge = functools.partial(
    pl.kernel,
    out_type=jax.ShapeDtypeStruct((NC, N, D), jnp.float32),
    mesh=plsc.VectorSubcoreMesh(core_axis_name="c", subcore_axis_name="s",
                                num_cores=NC, num_subcores=NS),
    compiler_params=pltpu.CompilerParams(needs_layout_passes=False),
    scratch_types=[
        pltpu.VMEM_SHARED((N, D), jnp.float32),
        pltpu.VMEM((W3,), jnp.int32),
        pltpu.VMEM((W3,), jnp.int32),
        pltpu.VMEM((CH,), jnp.int32),
        pltpu.VMEM((CH,), jnp.int32),
        pltpu.VMEM((CH,), jnp.int32),
        pltpu.VMEM((CH,), jnp.int32),
        pltpu.VMEM((CH,), jnp.int32),
        pltpu.VMEM((CH,), jnp.int32),
        pltpu.VMEM((CH, D), jnp.float32),
        pltpu.VMEM((CH, D), jnp.float32),
        pltpu.VMEM((CH, D), jnp.float32),
        pltpu.VMEM((CH, D), jnp.float32),
        pltpu.VMEM((CH, D), jnp.float32),
        pltpu.VMEM((CH, D), jnp.float32),
        pltpu.VMEM((CH,), jnp.float32),
        pltpu.VMEM((4, H), jnp.float32),
        pltpu.SemaphoreType.DMA,
        pltpu.SemaphoreType.DMA,
        pltpu.SemaphoreType.DMA,
        pltpu.SemaphoreType.DMA,
        pltpu.SemaphoreType.DMA,
        pltpu.SemaphoreType.DMA,
        pltpu.SemaphoreType.DMA,
        pltpu.SemaphoreType.DMA,
        pltpu.SemaphoreType.DMA,
        pltpu.SemaphoreType.DMA,
    ],
)(_sc_edge_body)


def _pre0_body(x_ref, ai_ref, aj_ref, ba_ref, bv_ref, bb_ref,
               u_ref, v_ref, wl_ref):
    xb = x_ref[...]
    u = jnp.dot(xb, ai_ref[...], preferred_element_type=jnp.float32)
    v = jnp.dot(xb, aj_ref[...], preferred_element_type=jnp.float32)
    u_ref[...] = u
    v_ref[...] = v
    hl = jnp.maximum(u + v + ba_ref[...], 0.0)
    logit = jnp.sum(hl * bv_ref[...], axis=1, keepdims=True) + bb_ref[...]
    wl_ref[...] = 1.0 / (1.0 + jnp.exp(-logit))


def _pre_bn_body(x_ref, g_ref, gb_ref, ai_ref, aj_ref, ba_ref, bv_ref, bb_ref,
                 xb_ref, u_ref, v_ref, wl_ref):
    x = x_ref[...]
    m = jnp.mean(x, axis=0, keepdims=True)
    xm = x - m
    var = jnp.mean(xm * xm, axis=0, keepdims=True)
    xb = xm / jnp.sqrt(var + 1e-5) * g_ref[...] + gb_ref[...]
    xb_ref[...] = xb
    u = jnp.dot(xb, ai_ref[...], preferred_element_type=jnp.float32)
    v = jnp.dot(xb, aj_ref[...], preferred_element_type=jnp.float32)
    u_ref[...] = u
    v_ref[...] = v
    hl = jnp.maximum(u + v + ba_ref[...], 0.0)
    logit = jnp.sum(hl * bv_ref[...], axis=1, keepdims=True) + bb_ref[...]
    wl_ref[...] = 1.0 / (1.0 + jnp.exp(-logit))


def _post_body(xb_ref, p_ref, wl_ref, wu_ref, wlw_ref, b_ref, out_ref):
    xb = xb_ref[...]
    aggr = p_ref[0] + p_ref[1] + wl_ref[...] * xb
    out_ref[...] = jnp.maximum(
        jnp.dot(xb, wu_ref[...], preferred_element_type=jnp.float32)
        + jnp.dot(aggr, wlw_ref[...], preferred_element_type=jnp.float32)
        + b_ref[...], 0.0)


def _heads1_body(emb_ref, few_ref, feb_ref, w1_ref, b1_ref, w2_ref, b2_ref,
                 wf_ref, bf_ref, x_ref, f_ref, xm_ref):
    x = (jnp.dot(emb_ref[...], few_ref[...],
                 preferred_element_type=jnp.float32) + feb_ref[...])
    x_ref[...] = x
    h = jnp.maximum(jnp.dot(x, w1_ref[...],
                            preferred_element_type=jnp.float32) + b1_ref[...],
                    0.0)
    h = jnp.maximum(jnp.dot(h, w2_ref[...],
                            preferred_element_type=jnp.float32) + b2_ref[...],
                    0.0)
    f_ref[...] = (jnp.dot(h, wf_ref[...],
                          preferred_element_type=jnp.float32) + bf_ref[...])
    xm_ref[...] = jnp.mean(x, axis=0, keepdims=True)


def _heads2_body(x_ref, t_ref, w1b_ref, w2_ref, b2_ref, wf_ref, bf_ref,
                 s_ref):
    h1 = jnp.maximum(jnp.dot(x_ref[...], w1b_ref[...],
                             preferred_element_type=jnp.float32) + t_ref[...],
                     0.0)
    h2 = jnp.maximum(jnp.dot(h1, w2_ref[...],
                             preferred_element_type=jnp.float32) + b2_ref[...],
                     0.0)
    s_ref[...] = (jnp.dot(h2, wf_ref[...],
                          preferred_element_type=jnp.float32) + bf_ref[...])


def _call_tc(body, out_shapes, *args):
    return pl.pallas_call(
        body,
        out_shape=[jax.ShapeDtypeStruct(s, jnp.float32) for s in out_shapes],
    )(*args)


def _mlp_vec(p, h):
    for lp in p["layers"]:
        h = jnp.maximum(h @ lp["w"] + lp["b"], 0.0)
    return h @ p["final"]["w"] + p["final"]["b"]


def kernel(x, edge_index, edge_attr, params):
    src = edge_index[0]
    dst = edge_index[1]
    ea_bits = lax.bitcast_convert_type(edge_attr, jnp.int32)
    eidx = jnp.stack([src.reshape(NCHT, CH), dst.reshape(NCHT, CH),
                      ea_bits.reshape(NCHT, CH)], axis=1).reshape(-1)

    emb = x
    for li, p in enumerate(params["gnn"]):
        a = p["linA"]["w"]
        pad = jnp.zeros((D, D - H), jnp.float32)
        ai = jnp.concatenate([a[:D], pad], axis=1)
        aj = jnp.concatenate([a[D:2 * D], pad], axis=1)
        zpad = jnp.zeros((1, D - H), jnp.float32)
        ba = jnp.concatenate([p["linA"]["b"].reshape(1, H), zpad], axis=1)
        bv = jnp.concatenate([p["linB"]["w"].reshape(1, H), zpad], axis=1)
        bb = p["linB"]["b"].reshape(1, 1)
        prm = jnp.stack([a[2 * D], p["linA"]["b"], p["linB"]["w"][:, 0],
                         jnp.full((H,), p["linB"]["b"][0], jnp.float32)])
        if li == 0:
            xb = emb
            u, v, wl = _call_tc(_pre0_body, [(N, D), (N, D), (N, 1)],
                                xb, ai, aj, ba, bv, bb)
        else:
            g = p["bn"]["g"].reshape(1, D)
            gb = p["bn"]["b"].reshape(1, D)
            xb, u, v, wl = _call_tc(_pre_bn_body,
                                    [(N, D), (N, D), (N, D), (N, 1)],
                                    emb, g, gb, ai, aj, ba, bv, bb)
        partials = _sc_edge(eidx, u, v, xb, prm)
        w = p["lin"]["w"]
        emb, = _call_tc(_post_body, [(N, D)],
                        xb, partials, wl, w[:D], w[D:], p["lin"]["b"].reshape(1, D))

    mf = params["mf"]
    X, f, xm = _call_tc(
        _heads1_body, [(N, D), (N, 1), (1, D)],
        emb, params["final_emb"]["w"], params["final_emb"]["b"].reshape(1, D),
        mf["layers"][0]["w"], mf["layers"][0]["b"].reshape(1, D),
        mf["layers"][1]["w"], mf["layers"][1]["b"].reshape(1, D),
        mf["final"]["w"], mf["final"]["b"].reshape(1, 1))
    f = f[:, 0]

    key = jax.random.key(123)
    p1v = jax.nn.softmax(f, axis=0)
    p1v = p1v * jnp.ones((N,), jnp.float32).at[N - 9:].set(0.0)
    a1 = jax.random.categorical(jax.random.fold_in(key, 0),
                                jnp.log(p1v)).astype(jnp.int32)
    p1 = p1v[a1]

    ms = params["ms"]
    xa1 = X[a1]
    t = (xa1 @ ms["layers"][0]["w"][:D] + ms["layers"][0]["b"]).reshape(1, D)
    s, = _call_tc(_heads2_body, [(N, 1)],
                  X, t, ms["layers"][0]["w"][D:],
                  ms["layers"][1]["w"], ms["layers"][1]["b"].reshape(1, D),
                  ms["final"]["w"], ms["final"]["b"].reshape(1, 1))
    s = s[:, 0]
    p2v = jax.nn.softmax(s, axis=0).at[a1].set(0.0)
    a2 = jax.random.categorical(jax.random.fold_in(key, 1),
                                jnp.log(p2v)).astype(jnp.int32)
    p2 = p2v[a2]

    xc = jnp.concatenate([xa1, X[a2]], axis=0)
    el = _mlp_vec(params["me"], xc)
    epv = jax.nn.softmax(el, axis=0)
    ae = jax.random.categorical(jax.random.fold_in(key, 2),
                                jnp.log(epv)).astype(jnp.int32)
    pe = epv[ae]

    pstop = 1.0 / (1.0 + jnp.exp(-_mlp_vec(params["mt"], xm[0])))
    pstop = pstop[0]
    ast = jax.random.bernoulli(jax.random.fold_in(key, 3),
                               pstop).astype(jnp.int32)
    ps = jnp.where(ast == 0, 1.0 - pstop, pstop)

    acts = jnp.stack([a1, a2, ae, ast]).astype(jnp.int32)
    probs = jnp.stack([p1, p2, pe, ps])
    return acts, probs
